# baseline (speedup 1.0000x reference)
import functools

import jax
import jax.numpy as jnp
from jax import lax
from jax.experimental import pallas as pl
from jax.experimental.pallas import tpu as pltpu
from jax.experimental.pallas import tpu_sc as plsc

B = 16384
NCF = 13
NS = 26
V = 100000
D = 32
OUT_W = NCF + NS * D
CAT_W = NS * D

_info = plsc.get_sparse_core_info()
NCORES = _info.num_cores
NSUB = _info.num_subcores
LANES = _info.num_lanes
NW = NCORES * NSUB
RPW = B // NW

CB = 64
NCH = RPW // CB
NIDX = CB * NS
GL = 128
NG = NIDX // GL

AB = 128
ANCH = RPW // AB
SB = 16
EROWS = SB * CAT_W // 128

_mesh = plsc.VectorSubcoreMesh(core_axis_name="c", subcore_axis_name="s")


@functools.partial(
    pl.kernel,
    mesh=_mesh,
    compiler_params=pltpu.CompilerParams(
        use_tc_tiling_on_sc=False, needs_layout_passes=False),
    out_type=jax.ShapeDtypeStruct((B * NS, D), jnp.float32),
    scratch_types=[
        pltpu.VMEM((CB, NS), jnp.int32),
        pltpu.VMEM((NG, GL), jnp.int32),
        pltpu.VMEM((NIDX,), jnp.int32),
        pltpu.VMEM((NIDX,), jnp.int32),
        pltpu.VMEM((NIDX,), jnp.int32),
        pltpu.VMEM((NIDX, D), jnp.float32),
        pltpu.SemaphoreType.DMA,
    ],
)
def _gather(cat_hbm, tab_hbm, out_hbm, catv, idxf, offp, gr, gc, gbuf, gsem):
    wid = lax.axis_index("s") * NCORES + lax.axis_index("c")
    row0 = wid * RPW
    iota = lax.iota(jnp.int32, LANES)

    for k in range(NIDX // LANES):
        p = iota + k * LANES
        s = p - (p // NS) * NS
        offp[pl.ds(k * LANES, LANES)] = s * V
        gr[pl.ds(k * LANES, LANES)] = p // NS
        gc[pl.ds(k * LANES, LANES)] = s

    def chunk_body(g, carry):
        base = row0 + g * CB
        pltpu.sync_copy(cat_hbm.at[pl.ds(base, CB), :], catv)
        for k in range(NIDX // LANES):
            sl = pl.ds(k * LANES, LANES)
            vals = plsc.load_gather(catv, [gr[sl], gc[sl]])
            idxf[k // 8, pl.ds((k % 8) * LANES, LANES)] = vals + offp[sl]
        cps = [
            pltpu.async_copy(
                tab_hbm.at[idxf.at[j]],
                gbuf.at[pl.ds(j * GL, GL), :],
                gsem)
            for j in range(NG)
        ]
        for cp in cps:
            cp.wait()
        pltpu.sync_copy(gbuf, out_hbm.at[pl.ds(base * NS, NIDX), :])
        return carry

    lax.fori_loop(0, NCH, chunk_body, 0)


@functools.partial(
    pl.kernel,
    mesh=_mesh,
    compiler_params=pltpu.CompilerParams(
        use_tc_tiling_on_sc=True, needs_layout_passes=False),
    out_type=jax.ShapeDtypeStruct((OUT_W, B), jnp.float32),
    scratch_types=[
        pltpu.VMEM((EROWS, 128), jnp.float32),
        pltpu.VMEM((SB, 128), jnp.float32),
        pltpu.VMEM((OUT_W, AB), jnp.float32),
        pltpu.SemaphoreType.DMA,
        pltpu.SemaphoreType.DMA,
    ],
)
def _assemble(emb_hbm, xc_hbm, out_hbm, ebuf, xbuf, obuf, esem, xsem):
    wid = lax.axis_index("s") * NCORES + lax.axis_index("c")
    row0 = wid * RPW
    iota = lax.iota(jnp.int32, LANES)

    def chunk_body(g, carry):
        base = pl.multiple_of(row0 + g * AB, AB)

        def sub_body(sub, carry1):
            sbase = pl.multiple_of(base + sub * SB, SB)
            erow = pl.multiple_of(sbase * CAT_W // 128, SB * CAT_W // 128)
            ecp = pltpu.async_copy(
                emb_hbm.at[pl.ds(erow, EROWS), :], ebuf, esem)
            xcp = pltpu.async_copy(xc_hbm.at[pl.ds(sbase, SB), :], xbuf, xsem)
            ecp.wait()
            xcp.wait()

            def row_body(r, carry2):
                bcol = iota * 0 + (sub * SB + r)
                fv = plsc.load_gather(xbuf, [iota * 0 + r, iota],
                                      mask=iota < NCF)
                plsc.store_scatter(obuf, [iota, bcol], fv, mask=iota < NCF)
                for k in range(CAT_W // LANES):
                    w = r * CAT_W + k * LANES
                    v = plsc.load_gather(
                        ebuf, [iota * 0 + w // 128, iota + (w % 128)])
                    plsc.store_scatter(
                        obuf, [iota + (NCF + k * LANES), bcol], v)
                return carry2

            lax.fori_loop(0, SB, row_body, 0)
            return carry1

        lax.fori_loop(0, AB // SB, sub_body, 0)
        pltpu.sync_copy(obuf, out_hbm.at[:, pl.ds(base, AB)])
        return carry

    lax.fori_loop(0, ANCH, chunk_body, 0)


def kernel(x_continuous, x_categorical, tables):
    cat = x_categorical.astype(jnp.int32)
    tab = tables.reshape(NS * V, D)
    emb = _gather(cat, tab)
    xcp = jnp.pad(x_continuous, ((0, 0), (0, 128 - NCF)))
    return _assemble(emb.reshape(B * CAT_W // 128, 128), xcp).T


# speedup vs baseline: 1.1333x; 1.1333x over previous
import functools

import jax
import jax.numpy as jnp
from jax import lax
from jax.experimental import pallas as pl
from jax.experimental.pallas import tpu as pltpu
from jax.experimental.pallas import tpu_sc as plsc

B = 16384
NCF = 13
NS = 26
V = 100000
D = 32
OUT_W = NCF + NS * D
CAT_W = NS * D

_info = plsc.get_sparse_core_info()
NCORES = _info.num_cores
NSUB = _info.num_subcores
LANES = _info.num_lanes
NW = NCORES * NSUB
RPW = B // NW

CB = 64
NCH = RPW // CB
NIDX = CB * NS
GL = 128
NG = NIDX // GL

AB = 128
ANCH = RPW // AB
SB = 16
EROWS = SB * CAT_W // 128

_mesh = plsc.VectorSubcoreMesh(core_axis_name="c", subcore_axis_name="s")


@functools.partial(
    pl.kernel,
    mesh=_mesh,
    compiler_params=pltpu.CompilerParams(
        use_tc_tiling_on_sc=False, needs_layout_passes=False),
    out_type=jax.ShapeDtypeStruct((B * NS, D), jnp.float32),
    scratch_types=[
        pltpu.VMEM((CB, NS), jnp.int32),
        pltpu.VMEM((NG, GL), jnp.int32),
        pltpu.VMEM((NIDX,), jnp.int32),
        pltpu.VMEM((NIDX,), jnp.int32),
        pltpu.VMEM((NIDX,), jnp.int32),
        pltpu.VMEM((NIDX, D), jnp.float32),
        pltpu.SemaphoreType.DMA,
    ],
)
def _gather(cat_hbm, tab_hbm, out_hbm, catv, idxf, offp, gr, gc, gbuf, gsem):
    wid = lax.axis_index("s") * NCORES + lax.axis_index("c")
    row0 = wid * RPW
    iota = lax.iota(jnp.int32, LANES)

    for k in range(NIDX // LANES):
        p = iota + k * LANES
        s = p - (p // NS) * NS
        offp[pl.ds(k * LANES, LANES)] = s * V
        gr[pl.ds(k * LANES, LANES)] = p // NS
        gc[pl.ds(k * LANES, LANES)] = s

    def chunk_body(g, carry):
        base = row0 + g * CB
        pltpu.sync_copy(cat_hbm.at[pl.ds(base, CB), :], catv)
        for k in range(NIDX // LANES):
            sl = pl.ds(k * LANES, LANES)
            vals = plsc.load_gather(catv, [gr[sl], gc[sl]])
            idxf[k // 8, pl.ds((k % 8) * LANES, LANES)] = vals + offp[sl]
        cps = [
            pltpu.async_copy(
                tab_hbm.at[idxf.at[j]],
                gbuf.at[pl.ds(j * GL, GL), :],
                gsem)
            for j in range(NG)
        ]
        for cp in cps:
            cp.wait()
        pltpu.sync_copy(gbuf, out_hbm.at[pl.ds(base * NS, NIDX), :])
        return carry

    lax.fori_loop(0, NCH, chunk_body, 0)


def kernel(x_continuous, x_categorical, tables):
    cat = x_categorical.astype(jnp.int32)
    tab = tables.reshape(NS * V, D)
    emb = _gather(cat, tab)
    return jnp.concatenate([x_continuous, emb.reshape(B, CAT_W)], axis=-1)
